# baseline (speedup 1.0000x reference)
import jax
import jax.numpy as jnp
from jax import lax
from jax.experimental import pallas as pl
from jax.experimental.pallas import tpu as pltpu
from jax.experimental.pallas import tpu_sc as plsc

_N = 646
_E = 20672
_NCOL = 648
_NP = 648
_NROW = 704

_NTILE = 16
_ROWS_T = _NROW // _NTILE
_Z = _ROWS_T * _NCOL
_ZPAD = 28544


def _sc_scatter_body(flat_hbm, out_hbm, flat_v, bacc_v, sem):
    s = lax.axis_index("s")
    w = lax.axis_index("c") * 16 + s
    load = pltpu.async_copy(flat_hbm, flat_v, sem)

    zeros16 = jnp.zeros((16,), jnp.float32)

    @plsc.parallel_loop(0, _ZPAD // 16, unroll=8)
    def _zero_body(i):
        bacc_v[pl.ds(i * 16, 16)] = zeros16

    load.wait()

    ones16 = jnp.ones((16,), jnp.float32)
    base = jnp.int32(w * _Z)
    zbound = jnp.uint32(_Z)

    @plsc.parallel_loop(0, _E // 16, unroll=4)
    def _scatter_body(t):
        loc = flat_v[pl.ds(t * 16, 16)] - base
        valid = plsc.bitcast(loc, jnp.uint32) < zbound
        plsc.addupdate_scatter(bacc_v, [loc], ones16, mask=valid)

    pltpu.sync_copy(bacc_v.at[pl.ds(0, _Z)], out_hbm.at[pl.ds(w * _Z, _Z)])


def _make_sc_scatter():
    return pl.kernel(
        _sc_scatter_body,
        out_type=jax.ShapeDtypeStruct((_NROW * _NCOL,), jnp.float32),
        mesh=plsc.VectorSubcoreMesh(core_axis_name="c", subcore_axis_name="s",
                                    num_cores=1),
        compiler_params=pltpu.CompilerParams(needs_layout_passes=False),
        scratch_types=[
            pltpu.VMEM((_E,), jnp.int32),
            pltpu.VMEM((_ZPAD,), jnp.float32),
            pltpu.SemaphoreType.DMA,
        ],
    )


def _tc_pre_body(x_ref, w1_ref, wlp_ref, blp_ref, h1_ref, res_ref):
    x = x_ref[...]
    pad2 = jnp.zeros((_NP - _N, 64), jnp.float32)
    h1 = jnp.dot(x, w1_ref[...], preferred_element_type=jnp.float32)
    h1_ref[...] = jnp.concatenate([h1, pad2])
    res = jnp.dot(x, wlp_ref[...], preferred_element_type=jnp.float32)
    res_ref[...] = jnp.concatenate([res + blp_ref[...], pad2])


def _tc_main_body(p_ref, x_ref, w1_ref, wlp_ref, blp_ref, b1_ref, w2_ref,
                  b2_ref, out_ref):
    x = x_ref[...]
    pad2 = jnp.zeros((_NP - _N, 64), jnp.float32)
    h1 = jnp.concatenate(
        [jnp.dot(x, w1_ref[...], preferred_element_type=jnp.float32), pad2])
    res = jnp.concatenate(
        [jnp.dot(x, wlp_ref[...], preferred_element_type=jnp.float32)
         + blp_ref[...], pad2])

    bmat = p_ref[pl.ds(0, _NP), :]
    rows = lax.broadcasted_iota(jnp.int32, (_NP, _NCOL), 0)
    cols = lax.broadcasted_iota(jnp.int32, (_NP, _NCOL), 1)
    eye = jnp.where((rows == cols) & (rows < _N), 1.0, 0.0)
    bmat = bmat + eye
    deg = jnp.sum(bmat, axis=1, keepdims=True)
    dinv = lax.rsqrt(jnp.maximum(deg, 1e-12))

    agg1 = dinv * jnp.dot(bmat, dinv * h1,
                          preferred_element_type=jnp.float32) + b1_ref[...]
    h = res + agg1
    h = h * (1.0 / (1.0 + jnp.exp(-h)))

    rmask = lax.broadcasted_iota(jnp.int32, (_NP, 64), 0) < _N
    cnt = float(_N * 64)
    mu = jnp.sum(jnp.where(rmask, h, 0.0)) / cnt
    dev = jnp.where(rmask, h - mu, 0.0)
    var = jnp.sum(dev * dev) / cnt
    hn = (h - mu) * lax.rsqrt(var + 1e-5)

    h2 = jnp.dot(hn, w2_ref[...], preferred_element_type=jnp.float32)
    out_ref[...] = dinv * jnp.dot(bmat, dinv * h2,
                                  preferred_element_type=jnp.float32) + b2_ref[...]


def kernel(x, edge_index, W1, b1, W2, b2, Wlp, blp):
    ei = edge_index.astype(jnp.int32)
    flat = ei[1] * _NCOL + ei[0]

    p = _make_sc_scatter()(flat).reshape(_NROW, _NCOL)

    out = pl.pallas_call(
        _tc_main_body,
        out_shape=jax.ShapeDtypeStruct((_NP, 64), jnp.float32),
    )(p, x, W1, Wlp, blp.reshape(1, -1), b1.reshape(1, -1), W2,
      b2.reshape(1, -1))
    return out[:_N]


# speedup vs baseline: 1.0082x; 1.0082x over previous
import jax
import jax.numpy as jnp
from jax import lax
from jax.experimental import pallas as pl
from jax.experimental.pallas import tpu as pltpu
from jax.experimental.pallas import tpu_sc as plsc

_N = 646
_E = 20672
_NCOL = 648
_NP = 648
_NROW = 672

_NTILE = 16
_ROWS_T = _NROW // _NTILE
_Z = _ROWS_T * _NCOL
_ZPAD = 27264
_EPAD = 20736


def _sc_scatter_body(flat_hbm, out_hbm, flat_v, bacc_v, sem):
    s = lax.axis_index("s")
    w = lax.axis_index("c") * 16 + s
    load = pltpu.async_copy(flat_hbm, flat_v.at[pl.ds(0, _E)], sem)

    zeros16 = jnp.zeros((16,), jnp.float32)

    @plsc.parallel_loop(0, _ZPAD // 16, unroll=8)
    def _zero_body(i):
        bacc_v[pl.ds(i * 16, 16)] = zeros16

    load.wait()
    poison16 = jnp.full((16,), jnp.int32(2**31 - 1))
    for t in range(_E // 16, _EPAD // 16):
        flat_v[pl.ds(t * 16, 16)] = poison16

    ones16 = jnp.ones((16,), jnp.float32)
    base = jnp.int32(w * _Z)
    zbound = jnp.uint32(_Z)

    @plsc.parallel_loop(0, _EPAD // 16, unroll=8)
    def _scatter_body(t):
        loc = flat_v[pl.ds(t * 16, 16)] - base
        valid = plsc.bitcast(loc, jnp.uint32) < zbound
        plsc.addupdate_scatter(bacc_v, [loc], ones16, mask=valid)

    pltpu.sync_copy(bacc_v.at[pl.ds(0, _Z)], out_hbm.at[pl.ds(w * _Z, _Z)])


def _make_sc_scatter():
    return pl.kernel(
        _sc_scatter_body,
        out_type=jax.ShapeDtypeStruct((_NROW * _NCOL,), jnp.float32),
        mesh=plsc.VectorSubcoreMesh(core_axis_name="c", subcore_axis_name="s",
                                    num_cores=1),
        compiler_params=pltpu.CompilerParams(needs_layout_passes=False),
        scratch_types=[
            pltpu.VMEM((_EPAD,), jnp.int32),
            pltpu.VMEM((_ZPAD,), jnp.float32),
            pltpu.SemaphoreType.DMA,
        ],
    )


def _tc_pre_body(x_ref, w1_ref, wlp_ref, blp_ref, h1_ref, res_ref):
    x = x_ref[...]
    pad2 = jnp.zeros((_NP - _N, 64), jnp.float32)
    h1 = jnp.dot(x, w1_ref[...], preferred_element_type=jnp.float32)
    h1_ref[...] = jnp.concatenate([h1, pad2])
    res = jnp.dot(x, wlp_ref[...], preferred_element_type=jnp.float32)
    res_ref[...] = jnp.concatenate([res + blp_ref[...], pad2])


def _tc_main_body(p_ref, h1_ref, res_ref, b1_ref, w2_ref, b2_ref, out_ref):
    bmat = p_ref[pl.ds(0, _NP), :]
    rows = lax.broadcasted_iota(jnp.int32, (_NP, _NCOL), 0)
    cols = lax.broadcasted_iota(jnp.int32, (_NP, _NCOL), 1)
    eye = jnp.where((rows == cols) & (rows < _N), 1.0, 0.0)
    bmat = bmat + eye
    deg = jnp.sum(bmat, axis=1, keepdims=True)
    dinv = lax.rsqrt(jnp.maximum(deg, 1e-12))

    agg1 = dinv * jnp.dot(bmat, dinv * h1_ref[...],
                          preferred_element_type=jnp.float32) + b1_ref[...]
    h = res_ref[...] + agg1
    h = h * (1.0 / (1.0 + jnp.exp(-h)))

    rmask = lax.broadcasted_iota(jnp.int32, (_NP, 64), 0) < _N
    cnt = float(_N * 64)
    mu = jnp.sum(jnp.where(rmask, h, 0.0)) / cnt
    dev = jnp.where(rmask, h - mu, 0.0)
    var = jnp.sum(dev * dev) / cnt
    hn = (h - mu) * lax.rsqrt(var + 1e-5)

    h2 = jnp.dot(hn, w2_ref[...], preferred_element_type=jnp.float32)
    out_ref[...] = dinv * jnp.dot(bmat, dinv * h2,
                                  preferred_element_type=jnp.float32) + b2_ref[...]


def kernel(x, edge_index, W1, b1, W2, b2, Wlp, blp):
    ei = edge_index.astype(jnp.int32)
    flat = ei[1] * _NCOL + ei[0]

    p = _make_sc_scatter()(flat).reshape(_NROW, _NCOL)

    h1p, resp = pl.pallas_call(
        _tc_pre_body,
        out_shape=(jax.ShapeDtypeStruct((_NP, 64), jnp.float32),
                   jax.ShapeDtypeStruct((_NP, 64), jnp.float32)),
    )(x, W1, Wlp, blp.reshape(1, -1))

    out = pl.pallas_call(
        _tc_main_body,
        out_shape=jax.ShapeDtypeStruct((_NP, 64), jnp.float32),
    )(p, h1p, resp, b1.reshape(1, -1), W2, b2.reshape(1, -1))
    return out[:_N]


# speedup vs baseline: 1.0090x; 1.0009x over previous
import jax
import jax.numpy as jnp
from jax import lax
from jax.experimental import pallas as pl
from jax.experimental.pallas import tpu as pltpu
from jax.experimental.pallas import tpu_sc as plsc

_N = 646
_E = 20672
_NCOL = 648
_NP = 648
_NROW = 672

_NTILE = 16
_ROWS_T = _NROW // _NTILE
_Z = _ROWS_T * _NCOL
_ZPAD = 27264
_EPAD = 20736


def _sc_scatter_body(flat_hbm, out_hbm, flat_v, bacc_v, sem):
    s = lax.axis_index("s")
    w = lax.axis_index("c") * 16 + s
    load = pltpu.async_copy(flat_hbm, flat_v.at[pl.ds(0, _E)], sem)

    zeros16 = jnp.zeros((16,), jnp.float32)

    @plsc.parallel_loop(0, _ZPAD // 16, unroll=8)
    def _zero_body(i):
        bacc_v[pl.ds(i * 16, 16)] = zeros16

    load.wait()
    poison16 = jnp.full((16,), jnp.int32(2**31 - 1))
    for t in range(_E // 16, _EPAD // 16):
        flat_v[pl.ds(t * 16, 16)] = poison16

    ones16 = jnp.ones((16,), jnp.float32)
    base = jnp.int32(w * _Z)
    zbound = jnp.uint32(_Z)

    @plsc.parallel_loop(0, _EPAD // 16, unroll=8)
    def _scatter_body(t):
        loc = flat_v[pl.ds(t * 16, 16)] - base
        valid = plsc.bitcast(loc, jnp.uint32) < zbound
        plsc.addupdate_scatter(bacc_v, [loc], ones16, mask=valid)

    pltpu.sync_copy(bacc_v.at[pl.ds(0, _Z)], out_hbm.at[pl.ds(w * _Z, _Z)])


def _make_sc_scatter():
    return pl.kernel(
        _sc_scatter_body,
        out_type=jax.ShapeDtypeStruct((_NROW * _NCOL,), jnp.float32),
        mesh=plsc.VectorSubcoreMesh(core_axis_name="c", subcore_axis_name="s",
                                    num_cores=1),
        compiler_params=pltpu.CompilerParams(needs_layout_passes=False),
        scratch_types=[
            pltpu.VMEM((_EPAD,), jnp.int32),
            pltpu.VMEM((_ZPAD,), jnp.float32),
            pltpu.SemaphoreType.DMA,
        ],
    )


def _tc_pre_body(x_ref, w1_ref, wlp_ref, blp_ref, h1_ref, res_ref):
    x = x_ref[...]
    pad2 = jnp.zeros((_NP - _N, 64), jnp.float32)
    h1 = jnp.dot(x, w1_ref[...], preferred_element_type=jnp.float32)
    h1_ref[...] = jnp.concatenate([h1, pad2])
    res = jnp.dot(x, wlp_ref[...], preferred_element_type=jnp.float32)
    res_ref[...] = jnp.concatenate([res + blp_ref[...], pad2])


def _tc_main_body(p_ref, h1_ref, res_ref, b1_ref, w2_ref, b2_ref, out_ref):
    bmat = p_ref[pl.ds(0, _NP), :]
    rows = lax.broadcasted_iota(jnp.int32, (_NP, _NCOL), 0)
    cols = lax.broadcasted_iota(jnp.int32, (_NP, _NCOL), 1)
    eye = jnp.where((rows == cols) & (rows < _N), 1.0, 0.0)
    bmat = bmat + eye
    deg = jnp.sum(bmat, axis=1, keepdims=True)
    dinv = lax.rsqrt(jnp.maximum(deg, 1e-12))

    agg1 = dinv * jnp.dot(bmat, dinv * h1_ref[...],
                          preferred_element_type=jnp.float32) + b1_ref[...]
    h = res_ref[...] + agg1
    h = h * (1.0 / (1.0 + jnp.exp(-h)))

    rmask = lax.broadcasted_iota(jnp.int32, (_NP, 64), 0) < _N
    cnt = float(_N * 64)
    mu = jnp.sum(jnp.where(rmask, h, 0.0)) / cnt
    dev = jnp.where(rmask, h - mu, 0.0)
    var = jnp.sum(dev * dev) / cnt
    hn = (h - mu) * lax.rsqrt(var + 1e-5)

    h2 = jnp.dot(hn, w2_ref[...], preferred_element_type=jnp.float32)
    out_ref[...] = dinv * jnp.dot(bmat, dinv * h2,
                                  preferred_element_type=jnp.float32) + b2_ref[...]


def kernel(x, edge_index, W1, b1, W2, b2, Wlp, blp):
    ei = edge_index.astype(jnp.int32)
    flat = ei[1] * _NCOL + ei[0]

    h1p, resp = pl.pallas_call(
        _tc_pre_body,
        out_shape=(jax.ShapeDtypeStruct((_NP, 64), jnp.float32),
                   jax.ShapeDtypeStruct((_NP, 64), jnp.float32)),
    )(x, W1, Wlp, blp.reshape(1, -1))

    p = _make_sc_scatter()(flat).reshape(_NROW, _NCOL)

    out = pl.pallas_call(
        _tc_main_body,
        out_shape=jax.ShapeDtypeStruct((_NP, 64), jnp.float32),
    )(p, h1p, resp, b1.reshape(1, -1), W2, b2.reshape(1, -1))
    return out[:_N]
